# Initial kernel scaffold; baseline (speedup 1.0000x reference)
import jax
import jax.numpy as jnp
from jax.experimental import pallas as pl


def kernel(xi, xj):
    raise NotImplementedError("write your pallas kernel here")


# speedup vs baseline: 9.0211x; 9.0211x over previous
import functools

import jax
import jax.numpy as jnp
from jax.experimental import pallas as pl

N = 4096
D = 128
RBLK = 256
K_RANK = N // 10
EPS = 1e-08
INF_KEY = 0x7F800000


def _sel_body(xi_ref, xjt_ref, loss_ref):
    blk = pl.program_id(0)
    xi = xi_ref[...]
    xjt = xjt_ref[...]
    a2 = jnp.sum(xi * xi, axis=1, keepdims=True)
    b2 = jnp.sum(xjt * xjt, axis=0, keepdims=True)
    prod = jax.lax.dot_general(
        xi, xjt, (((1,), (0,)), ((), ())),
        preferred_element_type=jnp.float32)
    d2 = a2 + b2 - 2.0 * prod
    rows = blk * RBLK + jax.lax.broadcasted_iota(jnp.int32, (RBLK, N), 0)
    cols = jax.lax.broadcasted_iota(jnp.int32, (RBLK, N), 1)
    masked = (cols == rows) | (cols == ((rows + N // 2) & (N - 1)))
    v = jnp.where(masked, -1.0, jnp.maximum(d2, 0.0))
    key = jax.lax.bitcast_convert_type(v, jnp.int32)

    lo = jnp.full((RBLK, 1), -1, jnp.int32)
    hi = jnp.full((RBLK, 1), INF_KEY, jnp.int32)

    def bs_body(_, carry):
        lo, hi = carry
        mid = lo + (hi - lo) // 2
        cnt = jnp.sum((key > mid).astype(jnp.int32), axis=1, keepdims=True)
        pred = cnt <= K_RANK
        return jnp.where(pred, lo, mid), jnp.where(pred, mid, hi)

    lo, hi = jax.lax.fori_loop(0, 31, bs_body, (lo, hi))
    kv = hi

    g = jnp.sum((key > kv).astype(jnp.int32), axis=1, keepdims=True)
    m1 = K_RANK - g + 1
    eq = key == kv
    lo2 = jnp.full((RBLK, 1), -1, jnp.int32)
    hi2 = jnp.full((RBLK, 1), N - 1, jnp.int32)

    def ix_body(_, carry):
        lo2, hi2 = carry
        mid = lo2 + (hi2 - lo2) // 2
        c2 = jnp.sum((eq & (cols <= mid)).astype(jnp.int32),
                     axis=1, keepdims=True)
        pred = c2 >= m1
        return jnp.where(pred, lo2, mid), jnp.where(pred, mid, hi2)

    _, jsel = jax.lax.fori_loop(0, 12, ix_body, (lo2, hi2))

    vk = jax.lax.bitcast_convert_type(kv, jnp.float32)
    sxi = jnp.sum(xi, axis=1, keepdims=True)
    sxj = jnp.sum(xjt, axis=0, keepdims=True)
    sel_sxj = jnp.sum(jnp.where(cols == jsel, sxj, 0.0),
                      axis=1, keepdims=True)
    dist = vk + 2.0 * EPS * (sxi - sel_sxj) + D * EPS * EPS
    loss_ref[...] = 1.0 / (dist + 1.0)


def _pairwise_losses(xi, xjt):
    return pl.pallas_call(
        _sel_body,
        grid=(N // RBLK,),
        in_specs=[
            pl.BlockSpec((RBLK, D), lambda i: (i, 0)),
            pl.BlockSpec((D, N), lambda i: (0, 0)),
        ],
        out_specs=pl.BlockSpec((RBLK, 1), lambda i: (i, 0)),
        out_shape=jax.ShapeDtypeStruct((N, 1), jnp.float32),
    )(xi, xjt)


@jax.jit
def kernel(xi, xj):
    losses = _pairwise_losses(xi, xj.T)
    return jnp.sum(losses) / N


# speedup vs baseline: 14.0639x; 1.5590x over previous
import functools

import jax
import jax.numpy as jnp
from jax.experimental import pallas as pl

N = 4096
D = 128
RBLK = 256
K_RANK = N // 10
EPS = 1e-08
INF_KEY = 0x7F800000


def _sel_body(xi_ref, xjt_ref, loss_ref):
    blk = pl.program_id(0)
    xi = xi_ref[...]
    xjt = xjt_ref[...]
    a2 = jnp.sum(xi * xi, axis=1, keepdims=True)
    b2 = jnp.sum(xjt * xjt, axis=0, keepdims=True)
    prod = jax.lax.dot_general(
        xi, xjt, (((1,), (0,)), ((), ())),
        preferred_element_type=jnp.float32)
    d2 = a2 + b2 - 2.0 * prod
    rows = blk * RBLK + jax.lax.broadcasted_iota(jnp.int32, (RBLK, N), 0)
    cols = jax.lax.broadcasted_iota(jnp.int32, (RBLK, N), 1)
    masked = (cols == rows) | (cols == ((rows + N // 2) & (N - 1)))
    v = jnp.where(masked, -1.0, jnp.maximum(d2, 0.0))
    key = jax.lax.bitcast_convert_type(v, jnp.int32)

    vmax = jnp.max(v, axis=1, keepdims=True)
    vmin = jnp.min(jnp.where(masked, jnp.inf, v), axis=1, keepdims=True)
    hi = jax.lax.bitcast_convert_type(vmax, jnp.int32) + 1
    lo = jax.lax.bitcast_convert_type(vmin, jnp.int32) - 1

    def bs_cond(carry):
        lo, hi = carry
        return jnp.any(hi - lo > 1)

    def bs_body(carry):
        lo, hi = carry
        mid = lo + (hi - lo) // 2
        cnt = jnp.sum((key > mid).astype(jnp.int32), axis=1, keepdims=True)
        pred = cnt <= K_RANK
        return jnp.where(pred, lo, mid), jnp.where(pred, mid, hi)

    lo, hi = jax.lax.while_loop(bs_cond, bs_body, (lo, hi))
    kv = hi

    jsel = jnp.min(jnp.where(key == kv, cols, N), axis=1, keepdims=True)

    vk = jax.lax.bitcast_convert_type(kv, jnp.float32)
    sxi = jnp.sum(xi, axis=1, keepdims=True)
    sxj = jnp.sum(xjt, axis=0, keepdims=True)
    sel_sxj = jnp.sum(jnp.where(cols == jsel, sxj, 0.0),
                      axis=1, keepdims=True)
    dist = vk + 2.0 * EPS * (sxi - sel_sxj) + D * EPS * EPS
    loss_ref[...] = 1.0 / (dist + 1.0)


def _pairwise_losses(xi, xjt):
    return pl.pallas_call(
        _sel_body,
        grid=(N // RBLK,),
        in_specs=[
            pl.BlockSpec((RBLK, D), lambda i: (i, 0)),
            pl.BlockSpec((D, N), lambda i: (0, 0)),
        ],
        out_specs=pl.BlockSpec((RBLK, 1), lambda i: (i, 0)),
        out_shape=jax.ShapeDtypeStruct((N, 1), jnp.float32),
    )(xi, xjt)


@jax.jit
def kernel(xi, xj):
    losses = _pairwise_losses(xi, xj.T)
    return jnp.sum(losses) / N
